# speedup vs baseline: 1.7953x; 1.7953x over previous
import jax
import jax.numpy as jnp
from jax.experimental import pallas as pl

LBLK = 512


def _bcast_kernel(table_ref, out_ref):
    blk = table_ref[...]
    out_ref[...] = jnp.broadcast_to(blk[None], out_ref.shape)


def kernel(inputs, table):
    b, l = inputs.shape
    d = table.shape[1]
    grid = (l // LBLK,)
    return pl.pallas_call(
        _bcast_kernel,
        grid=grid,
        in_specs=[pl.BlockSpec((LBLK, d), lambda i: (i, 0))],
        out_specs=pl.BlockSpec((b, LBLK, d), lambda i: (0, i, 0)),
        out_shape=jax.ShapeDtypeStruct((b, l, d), table.dtype),
    )(table[:l])


# speedup vs baseline: 2.6437x; 1.4726x over previous
import jax
import jax.numpy as jnp
from jax.experimental import pallas as pl

LBLK = 512


def _bcast_kernel(table_ref, out_ref):
    blk = table_ref[...]
    out_ref[...] = jnp.broadcast_to(blk[None], out_ref.shape)


def kernel(inputs, table):
    b, l = inputs.shape
    d = table.shape[1]
    grid = (l // LBLK,)
    return pl.pallas_call(
        _bcast_kernel,
        grid=grid,
        in_specs=[pl.BlockSpec((LBLK, d), lambda i: (i, 0))],
        out_specs=pl.BlockSpec((b, LBLK, d), lambda i: (0, i, 0)),
        out_shape=jax.ShapeDtypeStruct((b, l, d), table.dtype),
    )(table)


# speedup vs baseline: 2.7414x; 1.0369x over previous
import jax
import jax.numpy as jnp
from jax.experimental import pallas as pl

LBLK = 1024


def _bcast_kernel(table_ref, out_ref):
    blk = table_ref[...]
    out_ref[...] = jnp.broadcast_to(blk[None], out_ref.shape)


def kernel(inputs, table):
    b, l = inputs.shape
    d = table.shape[1]
    grid = (l // LBLK,)
    return pl.pallas_call(
        _bcast_kernel,
        grid=grid,
        in_specs=[pl.BlockSpec((LBLK, d), lambda i: (i, 0))],
        out_specs=pl.BlockSpec((b, LBLK, d), lambda i: (0, i, 0)),
        out_shape=jax.ShapeDtypeStruct((b, l, d), table.dtype),
    )(table)
